# baseline (speedup 1.0000x reference)
import functools

import jax
import jax.numpy as jnp
from jax import lax
from jax.experimental import pallas as pl
from jax.experimental.pallas import tpu as pltpu
from jax.experimental.pallas import tpu_sc as plsc

N = 10000
NP = 10240
E = 160000
DM = 256
HG = 128
NSC = 2
NSUB = 16
LANE = 16

C = 64
G = C // LANE
EPAD = 160768
EP = EPAD // NSUB
NCHUNK = EP // C
RN = NP // NSUB
RC = 80
NNORM = RN // RC

RB = 400
GRID = N // RB


def _qkv_body(x_ref, wq_ref, wk_ref, wv_ref, q_ref, kv_ref):
    xb = x_ref[...]
    qb = jnp.dot(xb, wq_ref[...], preferred_element_type=jnp.float32)
    kb = jnp.dot(xb, wk_ref[...], preferred_element_type=jnp.float32)
    vb = jnp.dot(xb, wv_ref[...], preferred_element_type=jnp.float32)
    q_ref[0] = qb[:, :HG]
    q_ref[1] = qb[:, HG:]
    kv_ref[0] = jnp.concatenate([kb[:, :HG], vb[:, :HG]], axis=1)
    kv_ref[1] = jnp.concatenate([kb[:, HG:], vb[:, HG:]], axis=1)


def _qkv(x, wq, wk, wv):
    return pl.pallas_call(
        _qkv_body,
        grid=(GRID,),
        in_specs=[
            pl.BlockSpec((RB, DM), lambda i: (i, 0)),
            pl.BlockSpec((DM, DM), lambda i: (0, 0)),
            pl.BlockSpec((DM, DM), lambda i: (0, 0)),
            pl.BlockSpec((DM, DM), lambda i: (0, 0)),
        ],
        out_specs=[
            pl.BlockSpec((NSC, RB, HG), lambda i: (0, i, 0)),
            pl.BlockSpec((NSC, RB, 2 * HG), lambda i: (0, i, 0)),
        ],
        out_shape=[
            jax.ShapeDtypeStruct((NSC, NP, HG), jnp.float32),
            jax.ShapeDtypeStruct((NSC, NP, 2 * HG), jnp.float32),
        ],
    )(x, wq, wk, wv)


def _edge_body(kv_hbm, q_hbm, src_hbm, dst_hbm, z144_hbm,
               out_hbm,
               acc,
               srcv, dstv, gsrcv, gdstv,
               kvrows, qrows, stage, sem):
    c = lax.axis_index("c")
    s = lax.axis_index("s")
    cN = c * NP

    r0 = s * RN
    pltpu.sync_copy(z144_hbm.at[pl.ds(r0, RN)], acc.at[pl.ds(r0, RN)])
    plsc.subcore_barrier()

    lanes = lax.iota(jnp.int32, LANE)
    base0 = s * EP
    zf = jnp.zeros((LANE,), jnp.float32)

    def dz(i, carry):
        stage[i, pl.ds(0, LANE)] = zf
        return carry
    lax.fori_loop(0, C, dz, 0)

    def chunk_body(t, carry):
        base = base0 + t * C
        pltpu.sync_copy(src_hbm.at[pl.ds(base, C)], srcv)
        pltpu.sync_copy(dst_hbm.at[pl.ds(base, C)], dstv)

        def gi(j, carry2):
            sl = pl.ds(j * LANE, LANE)
            gsrcv[sl] = srcv[sl] + cN
            gdstv[sl] = dstv[sl] + cN
            return carry2
        lax.fori_loop(0, C // LANE, gi, 0)

        cp1 = pltpu.async_copy(kv_hbm.at[gsrcv], kvrows, sem)
        cp2 = pltpu.async_copy(q_hbm.at[gdstv], qrows, sem)
        cp1.wait()
        cp2.wait()

        for g in range(G):
            ridx = lanes + (g * LANE)

            def dot_d(d, accs):
                cb = jnp.bitwise_and(d + lanes, 31)
                out = []
                for h in range(4):
                    kvec = (h * 32) + cb
                    kvv = plsc.load_gather(kvrows, [ridx, kvec])
                    qv = plsc.load_gather(qrows, [ridx, kvec])
                    out.append(accs[h] + kvv * qv)
                return tuple(out)
            accs = lax.fori_loop(0, 32, dot_d, (zf, zf, zf, zf), unroll=4)
            evs = []
            for h in range(4):
                ev = jnp.exp(accs[h] * 0.0625)
                evs.append(ev)
                plsc.store_scatter(stage, [ridx, jnp.full((LANE,), h, jnp.int32)], ev)

            def scale_d(d, carry2):
                cb = jnp.bitwise_and(d + lanes, 31)
                for h in range(4):
                    cvec = HG + (h * 32) + cb
                    svec = LANE + (h * 32) + cb
                    vv = plsc.load_gather(kvrows, [ridx, cvec])
                    plsc.store_scatter(stage, [ridx, svec], vv * evs[h])
                return carry2
            lax.fori_loop(0, 32, scale_d, 0, unroll=4)

        pltpu.sync_copy(stage, acc.at[dstv], add=True)
        return carry
    lax.fori_loop(0, NCHUNK, chunk_body, 0)
    plsc.subcore_barrier()

    pltpu.sync_copy(acc.at[pl.ds(r0, RN)], out_hbm.at[pl.ds(cN + r0, RN)])


def _make_edge_call():
    mesh = plsc.VectorSubcoreMesh(core_axis_name="c", subcore_axis_name="s")
    f32 = jnp.float32
    return pl.kernel(
        _edge_body,
        out_type=jax.ShapeDtypeStruct((NSC * NP, 144), f32),
        mesh=mesh,
        compiler_params=pltpu.CompilerParams(
            needs_layout_passes=False, use_tc_tiling_on_sc=False),
        scratch_types=[
            pltpu.VMEM_SHARED((NP, 144), f32),
            pltpu.VMEM((C,), jnp.int32),
            pltpu.VMEM((C,), jnp.int32),
            pltpu.VMEM((C,), jnp.int32),
            pltpu.VMEM((C,), jnp.int32),
            pltpu.VMEM((C, 2 * HG), f32),
            pltpu.VMEM((C, HG), f32),
            pltpu.VMEM((C, 144), f32),
            pltpu.SemaphoreType.DMA,
        ],
    )


_EDGE_CALL = _make_edge_call()


def _ln(x, g, b):
    mu = jnp.mean(x, axis=-1, keepdims=True)
    var = jnp.mean((x - mu) ** 2, axis=-1, keepdims=True)
    return (x - mu) / jnp.sqrt(var + 1e-5) * g + b


def _post_body(ft2_ref, x_ref, g_ref, b_ref, w1_ref, b1_ref, al_ref,
               w2_ref, b2_ref, o_ref):
    g = g_ref[...]
    b = b_ref[...]
    halves = []
    for cc in range(NSC):
        blk = ft2_ref[cc]
        den = jnp.maximum(blk[:, :4], 1e-20)
        denb = jnp.concatenate(
            [jnp.broadcast_to(den[:, hh:hh + 1], (RB, 32)) for hh in range(4)],
            axis=1)
        halves.append(blk[:, LANE:] / denb)
    rst = jnp.concatenate(halves, axis=1) + x_ref[...]
    rst = _ln(rst, g, b)
    h = jnp.dot(rst, w1_ref[...], preferred_element_type=jnp.float32)
    h = h + b1_ref[...]
    h = jnp.where(h > 0, h, al_ref[...] * h)
    ffn = jnp.dot(h, w2_ref[...], preferred_element_type=jnp.float32)
    ffn = ffn + b2_ref[...]
    o_ref[...] = _ln(rst + ffn, g, b)


def _post(ft2, x, g, b, w1, b1, al, w2, b2):
    d_ff = w1.shape[1]
    return pl.pallas_call(
        _post_body,
        grid=(GRID,),
        in_specs=[
            pl.BlockSpec((NSC, RB, 144), lambda i: (0, i, 0)),
            pl.BlockSpec((RB, DM), lambda i: (i, 0)),
            pl.BlockSpec((1, DM), lambda i: (0, 0)),
            pl.BlockSpec((1, DM), lambda i: (0, 0)),
            pl.BlockSpec((DM, d_ff), lambda i: (0, 0)),
            pl.BlockSpec((1, d_ff), lambda i: (0, 0)),
            pl.BlockSpec((1, d_ff), lambda i: (0, 0)),
            pl.BlockSpec((d_ff, DM), lambda i: (0, 0)),
            pl.BlockSpec((1, DM), lambda i: (0, 0)),
        ],
        out_specs=pl.BlockSpec((RB, DM), lambda i: (i, 0)),
        out_shape=jax.ShapeDtypeStruct((N, DM), jnp.float32),
    )(ft2, x, g, b, w1, b1, al, w2, b2)


def kernel(x, params, edge_index):
    src = jnp.concatenate(
        [edge_index[0], jnp.zeros((EPAD - E,), jnp.int32)])
    dst = jnp.concatenate(
        [edge_index[1], jnp.full((EPAD - E,), NP - 1, jnp.int32)])
    z144 = jnp.zeros((NP, 144), jnp.float32)
    feat = x
    for p in params:
        q, kv = _qkv(feat, p['Wq'], p['Wk'], p['Wv'])
        ft2 = _EDGE_CALL(
            kv.reshape(NSC * NP, 2 * HG), q.reshape(NSC * NP, HG),
            src, dst, z144)
        feat = _post(
            ft2.reshape(NSC, NP, 144), feat,
            p['g'].reshape(1, DM), p['b'].reshape(1, DM),
            p['W1'], p['b1'].reshape(1, -1), p['alpha'].reshape(1, -1),
            p['W2'], p['b2'].reshape(1, DM))
    return feat


# speedup vs baseline: 1.1050x; 1.1050x over previous
import functools

import jax
import jax.numpy as jnp
from jax import lax
from jax.experimental import pallas as pl
from jax.experimental.pallas import tpu as pltpu
from jax.experimental.pallas import tpu_sc as plsc

N = 10000
NP = 10240
E = 160000
DM = 256
HG = 128
NSC = 2
NSUB = 16
LANE = 16

C = 80
G = C // LANE
EP = E // NSUB
NCHUNK = EP // C
RN = NP // NSUB
RC = 80
NNORM = RN // RC

RB = 400
GRID = N // RB


def _qkv_body(x_ref, wq_ref, wk_ref, wv_ref, q_ref, kv_ref):
    xb = x_ref[...]
    qb = jnp.dot(xb, wq_ref[...], preferred_element_type=jnp.float32)
    kb = jnp.dot(xb, wk_ref[...], preferred_element_type=jnp.float32)
    vb = jnp.dot(xb, wv_ref[...], preferred_element_type=jnp.float32)
    q_ref[0] = qb[:, :HG]
    q_ref[1] = qb[:, HG:]
    kv_ref[0] = jnp.concatenate([kb[:, :HG], vb[:, :HG]], axis=1)
    kv_ref[1] = jnp.concatenate([kb[:, HG:], vb[:, HG:]], axis=1)


def _qkv(x, wq, wk, wv):
    return pl.pallas_call(
        _qkv_body,
        grid=(GRID,),
        in_specs=[
            pl.BlockSpec((RB, DM), lambda i: (i, 0)),
            pl.BlockSpec((DM, DM), lambda i: (0, 0)),
            pl.BlockSpec((DM, DM), lambda i: (0, 0)),
            pl.BlockSpec((DM, DM), lambda i: (0, 0)),
        ],
        out_specs=[
            pl.BlockSpec((NSC, RB, HG), lambda i: (0, i, 0)),
            pl.BlockSpec((NSC, RB, 2 * HG), lambda i: (0, i, 0)),
        ],
        out_shape=[
            jax.ShapeDtypeStruct((NSC, NP, HG), jnp.float32),
            jax.ShapeDtypeStruct((NSC, NP, 2 * HG), jnp.float32),
        ],
    )(x, wq, wk, wv)


def _edge_body(kv_hbm, q_hbm, src_hbm, dst_hbm, z128_hbm, z16_hbm,
               outf_hbm, outd_hbm,
               ft2_acc, den_acc,
               srcv, dstv, gsrcv, gdstv,
               kvrows, qrows, dstage, sem):
    c = lax.axis_index("c")
    s = lax.axis_index("s")
    cN = c * NP

    r0 = s * RN
    pltpu.sync_copy(z128_hbm.at[pl.ds(r0, RN)], ft2_acc.at[pl.ds(r0, RN)])
    pltpu.sync_copy(z16_hbm.at[pl.ds(r0, RN)], den_acc.at[pl.ds(r0, RN)])
    plsc.subcore_barrier()

    lanes = lax.iota(jnp.int32, LANE)
    base0 = s * EP

    def dz(i, carry):
        dstage[i] = jnp.zeros((LANE,), jnp.float32)
        return carry
    lax.fori_loop(0, C, dz, 0)

    def chunk_body(t, carry):
        base = base0 + t * C
        pltpu.sync_copy(src_hbm.at[pl.ds(base, C)], srcv)
        pltpu.sync_copy(dst_hbm.at[pl.ds(base, C)], dstv)

        def gi(j, carry2):
            sl = pl.ds(j * LANE, LANE)
            gsrcv[sl] = srcv[sl] + cN
            gdstv[sl] = dstv[sl] + cN
            return carry2
        lax.fori_loop(0, C // LANE, gi, 0)

        cp1 = pltpu.async_copy(kv_hbm.at[gsrcv], kvrows, sem)
        cp2 = pltpu.async_copy(q_hbm.at[gdstv], qrows, sem)
        cp1.wait()
        cp2.wait()

        zf = jnp.zeros((LANE,), jnp.float32)
        for g in range(G):
            ridx = lanes + (g * LANE)

            def dot_d(d, accs):
                cb = jnp.bitwise_and(d + lanes, 31)
                out = []
                for h in range(4):
                    cvec = (h * 32) + cb
                    kvv = plsc.load_gather(kvrows, [ridx, cvec])
                    qv = plsc.load_gather(qrows, [ridx, cvec])
                    out.append(accs[h] + kvv * qv)
                return tuple(out)
            accs = lax.fori_loop(0, 32, dot_d, (zf, zf, zf, zf), unroll=4)
            evs = []
            for h in range(4):
                ev = jnp.exp(accs[h] * 0.0625)
                evs.append(ev)
                plsc.store_scatter(dstage, [ridx, jnp.full((LANE,), h, jnp.int32)], ev)

            def scale_d(d, carry2):
                cb = jnp.bitwise_and(d + lanes, 31)
                for h in range(4):
                    cvec = (h * 32) + cb
                    vv = plsc.load_gather(kvrows, [ridx, HG + cvec])
                    plsc.store_scatter(qrows, [ridx, cvec], vv * evs[h])
                return carry2
            lax.fori_loop(0, 32, scale_d, 0, unroll=4)
        pltpu.sync_copy(qrows, ft2_acc.at[dstv], add=True)
        pltpu.sync_copy(dstage, den_acc.at[dstv], add=True)
        return carry
    lax.fori_loop(0, NCHUNK, chunk_body, 0)
    plsc.subcore_barrier()

    pltpu.sync_copy(ft2_acc.at[pl.ds(r0, RN)], outf_hbm.at[pl.ds(cN + r0, RN)])
    pltpu.sync_copy(den_acc.at[pl.ds(r0, RN)], outd_hbm.at[pl.ds(cN + r0, RN)])


def _make_edge_call():
    mesh = plsc.VectorSubcoreMesh(core_axis_name="c", subcore_axis_name="s")
    f32 = jnp.float32
    return pl.kernel(
        _edge_body,
        out_type=(jax.ShapeDtypeStruct((NSC * NP, HG), f32),
                  jax.ShapeDtypeStruct((NSC * NP, LANE), f32)),
        mesh=mesh,
        compiler_params=pltpu.CompilerParams(
            needs_layout_passes=False, use_tc_tiling_on_sc=False),
        scratch_types=[
            pltpu.VMEM_SHARED((NP, HG), f32),
            pltpu.VMEM_SHARED((NP, LANE), f32),
            pltpu.VMEM((C,), jnp.int32),
            pltpu.VMEM((C,), jnp.int32),
            pltpu.VMEM((C,), jnp.int32),
            pltpu.VMEM((C,), jnp.int32),
            pltpu.VMEM((C, 2 * HG), f32),
            pltpu.VMEM((C, HG), f32),
            pltpu.VMEM((C, LANE), f32),
            pltpu.SemaphoreType.DMA,
        ],
    )


_EDGE_CALL = _make_edge_call()


def _ln(x, g, b):
    mu = jnp.mean(x, axis=-1, keepdims=True)
    var = jnp.mean((x - mu) ** 2, axis=-1, keepdims=True)
    return (x - mu) / jnp.sqrt(var + 1e-5) * g + b


def _post_body(ft2_ref, den_ref, x_ref, g_ref, b_ref, w1_ref, b1_ref, al_ref,
               w2_ref, b2_ref, o_ref):
    g = g_ref[...]
    b = b_ref[...]
    halves = []
    for cc in range(NSC):
        den = jnp.maximum(den_ref[cc][:, :4], 1e-20)
        denb = jnp.concatenate(
            [jnp.broadcast_to(den[:, hh:hh + 1], (RB, 32)) for hh in range(4)],
            axis=1)
        halves.append(ft2_ref[cc] / denb)
    rst = jnp.concatenate(halves, axis=1) + x_ref[...]
    rst = _ln(rst, g, b)
    h = jnp.dot(rst, w1_ref[...], preferred_element_type=jnp.float32)
    h = h + b1_ref[...]
    h = jnp.where(h > 0, h, al_ref[...] * h)
    ffn = jnp.dot(h, w2_ref[...], preferred_element_type=jnp.float32)
    ffn = ffn + b2_ref[...]
    o_ref[...] = _ln(rst + ffn, g, b)


def _post(ft2, den, x, g, b, w1, b1, al, w2, b2):
    d_ff = w1.shape[1]
    return pl.pallas_call(
        _post_body,
        grid=(GRID,),
        in_specs=[
            pl.BlockSpec((NSC, RB, HG), lambda i: (0, i, 0)),
            pl.BlockSpec((NSC, RB, LANE), lambda i: (0, i, 0)),
            pl.BlockSpec((RB, DM), lambda i: (i, 0)),
            pl.BlockSpec((1, DM), lambda i: (0, 0)),
            pl.BlockSpec((1, DM), lambda i: (0, 0)),
            pl.BlockSpec((DM, d_ff), lambda i: (0, 0)),
            pl.BlockSpec((1, d_ff), lambda i: (0, 0)),
            pl.BlockSpec((1, d_ff), lambda i: (0, 0)),
            pl.BlockSpec((d_ff, DM), lambda i: (0, 0)),
            pl.BlockSpec((1, DM), lambda i: (0, 0)),
        ],
        out_specs=pl.BlockSpec((RB, DM), lambda i: (i, 0)),
        out_shape=jax.ShapeDtypeStruct((N, DM), jnp.float32),
    )(ft2, den, x, g, b, w1, b1, al, w2, b2)


def kernel(x, params, edge_index):
    src = edge_index[0]
    dst = edge_index[1]
    z128 = jnp.zeros((NP, HG), jnp.float32)
    z16 = jnp.zeros((NP, LANE), jnp.float32)
    feat = x
    for p in params:
        q, kv = _qkv(feat, p['Wq'], p['Wk'], p['Wv'])
        ft2, den = _EDGE_CALL(
            kv.reshape(NSC * NP, 2 * HG), q.reshape(NSC * NP, HG),
            src, dst, z128, z16)
        feat = _post(
            ft2.reshape(NSC, NP, HG), den.reshape(NSC, NP, LANE), feat,
            p['g'].reshape(1, DM), p['b'].reshape(1, DM),
            p['W1'], p['b1'].reshape(1, -1), p['alpha'].reshape(1, -1),
            p['W2'], p['b2'].reshape(1, DM))
    return feat


# speedup vs baseline: 1.1241x; 1.0173x over previous
import functools

import jax
import jax.numpy as jnp
from jax import lax
from jax.experimental import pallas as pl
from jax.experimental.pallas import tpu as pltpu
from jax.experimental.pallas import tpu_sc as plsc

N = 10000
NP = 10240
E = 160000
DM = 256
HG = 128
NSC = 2
NSUB = 16
LANE = 16

C = 80
G = C // LANE
EP = E // NSUB
NCHUNK = EP // C
RN = NP // NSUB
RC = 80
NNORM = RN // RC

RB = 400
GRID = N // RB


def _qkv_body(x_ref, wq_ref, wk_ref, wv_ref, q_ref, kv_ref):
    xb = x_ref[...]
    qb = jnp.dot(xb, wq_ref[...], preferred_element_type=jnp.float32)
    kb = jnp.dot(xb, wk_ref[...], preferred_element_type=jnp.float32)
    vb = jnp.dot(xb, wv_ref[...], preferred_element_type=jnp.float32)
    q_ref[0] = qb[:, :HG]
    q_ref[1] = qb[:, HG:]
    kv_ref[0] = jnp.concatenate([kb[:, :HG], vb[:, :HG]], axis=1)
    kv_ref[1] = jnp.concatenate([kb[:, HG:], vb[:, HG:]], axis=1)


def _qkv(x, wq, wk, wv):
    out = jax.ShapeDtypeStruct((NSC, NP, HG), jnp.float32)
    return pl.pallas_call(
        _qkv_body,
        grid=(GRID,),
        in_specs=[
            pl.BlockSpec((RB, DM), lambda i: (i, 0)),
            pl.BlockSpec((DM, DM), lambda i: (0, 0)),
            pl.BlockSpec((DM, DM), lambda i: (0, 0)),
            pl.BlockSpec((DM, DM), lambda i: (0, 0)),
        ],
        out_specs=[
            pl.BlockSpec((NSC, RB, HG), lambda i: (0, i, 0)),
            pl.BlockSpec((NSC, RB, 2 * HG), lambda i: (0, i, 0)),
        ],
        out_shape=[
            out,
            jax.ShapeDtypeStruct((NSC, NP, 2 * HG), jnp.float32),
        ],
    )(x, wq, wk, wv)


def _edge_body(kv_hbm, q_hbm, src_hbm, dst_hbm, z128_hbm, z16_hbm,
               out_hbm,
               ft2_acc, den_acc,
               srcv, dstv, gsrcv, gdstv,
               kvrows, qrows, dstage, sem):
    c = lax.axis_index("c")
    s = lax.axis_index("s")
    cN = c * NP

    r0 = s * RN
    pltpu.sync_copy(z128_hbm.at[pl.ds(r0, RN)], ft2_acc.at[pl.ds(r0, RN)])
    pltpu.sync_copy(z16_hbm.at[pl.ds(r0, RN)], den_acc.at[pl.ds(r0, RN)])
    plsc.subcore_barrier()

    lanes = lax.iota(jnp.int32, LANE)
    base0 = s * EP

    def dz(i, carry):
        dstage[i] = jnp.zeros((LANE,), jnp.float32)
        return carry
    lax.fori_loop(0, C, dz, 0)

    def chunk_body(t, carry):
        base = base0 + t * C
        pltpu.sync_copy(src_hbm.at[pl.ds(base, C)], srcv)
        pltpu.sync_copy(dst_hbm.at[pl.ds(base, C)], dstv)

        def gi(j, carry2):
            sl = pl.ds(j * LANE, LANE)
            gsrcv[sl] = srcv[sl] + cN
            gdstv[sl] = dstv[sl] + cN
            return carry2
        lax.fori_loop(0, C // LANE, gi, 0)

        cp1 = pltpu.async_copy(kv_hbm.at[gsrcv], kvrows, sem)
        cp2 = pltpu.async_copy(q_hbm.at[gdstv], qrows, sem)
        cp1.wait()
        cp2.wait()

        zf = jnp.zeros((LANE,), jnp.float32)
        for g in range(G):
            ridx = lanes + (g * LANE)

            def dot_d(d, accs):
                cb = jnp.bitwise_and(d + lanes, 31)
                out = []
                for h in range(4):
                    cvec = (h * 32) + cb
                    kvv = plsc.load_gather(kvrows, [ridx, cvec])
                    qv = plsc.load_gather(qrows, [ridx, cvec])
                    out.append(accs[h] + kvv * qv)
                return tuple(out)
            accs = lax.fori_loop(0, 32, dot_d, (zf, zf, zf, zf), unroll=4)
            evs = []
            for h in range(4):
                ev = jnp.exp(accs[h] * 0.0625)
                evs.append(ev)
                plsc.store_scatter(dstage, [ridx, jnp.full((LANE,), h, jnp.int32)], ev)

            def scale_d(d, carry2):
                cb = jnp.bitwise_and(d + lanes, 31)
                for h in range(4):
                    cvec = (h * 32) + cb
                    vv = plsc.load_gather(kvrows, [ridx, HG + cvec])
                    plsc.store_scatter(qrows, [ridx, cvec], vv * evs[h])
                return carry2
            lax.fori_loop(0, 32, scale_d, 0, unroll=4)
        pltpu.sync_copy(qrows, ft2_acc.at[dstv], add=True)
        pltpu.sync_copy(dstage, den_acc.at[dstv], add=True)
        return carry
    lax.fori_loop(0, NCHUNK, chunk_body, 0)
    plsc.subcore_barrier()

    def norm_chunk(t, carry):
        row = s * RN + t * RC
        pltpu.sync_copy(ft2_acc.at[pl.ds(row, RC)], qrows)
        pltpu.sync_copy(den_acc.at[pl.ds(row, RC)], dstage)

        def row_body(r, carry2):
            dv = jnp.maximum(dstage[r], 1e-20)
            inv = 1.0 / dv
            for h in range(4):
                bv = jnp.full((LANE,), inv[h], jnp.float32)
                for j in (2 * h, 2 * h + 1):
                    sl = pl.ds(j * LANE, LANE)
                    qrows[r, sl] = qrows[r, sl] * bv
            return carry2
        lax.fori_loop(0, RC, row_body, 0)
        pltpu.sync_copy(qrows, out_hbm.at[pl.ds(cN + row, RC)])
        return carry
    lax.fori_loop(0, NNORM, norm_chunk, 0)


def _make_edge_call():
    mesh = plsc.VectorSubcoreMesh(core_axis_name="c", subcore_axis_name="s")
    f32 = jnp.float32
    return pl.kernel(
        _edge_body,
        out_type=jax.ShapeDtypeStruct((NSC * NP, HG), f32),
        mesh=mesh,
        compiler_params=pltpu.CompilerParams(
            needs_layout_passes=False, use_tc_tiling_on_sc=False),
        scratch_types=[
            pltpu.VMEM_SHARED((NP, HG), f32),
            pltpu.VMEM_SHARED((NP, LANE), f32),
            pltpu.VMEM((C,), jnp.int32),
            pltpu.VMEM((C,), jnp.int32),
            pltpu.VMEM((C,), jnp.int32),
            pltpu.VMEM((C,), jnp.int32),
            pltpu.VMEM((C, 2 * HG), f32),
            pltpu.VMEM((C, HG), f32),
            pltpu.VMEM((C, LANE), f32),
            pltpu.SemaphoreType.DMA,
        ],
    )


_EDGE_CALL = _make_edge_call()


def _ln(x, g, b):
    mu = jnp.mean(x, axis=-1, keepdims=True)
    var = jnp.mean((x - mu) ** 2, axis=-1, keepdims=True)
    return (x - mu) / jnp.sqrt(var + 1e-5) * g + b


def _post_body(ft2_ref, x_ref, g_ref, b_ref, w1_ref, b1_ref, al_ref,
               w2_ref, b2_ref, o_ref):
    g = g_ref[...]
    b = b_ref[...]
    rst = jnp.concatenate([ft2_ref[0], ft2_ref[1]], axis=1) + x_ref[...]
    rst = _ln(rst, g, b)
    h = jnp.dot(rst, w1_ref[...], preferred_element_type=jnp.float32)
    h = h + b1_ref[...]
    h = jnp.where(h > 0, h, al_ref[...] * h)
    ffn = jnp.dot(h, w2_ref[...], preferred_element_type=jnp.float32)
    ffn = ffn + b2_ref[...]
    o_ref[...] = _ln(rst + ffn, g, b)


def _post(ft2, x, g, b, w1, b1, al, w2, b2):
    d_ff = w1.shape[1]
    return pl.pallas_call(
        _post_body,
        grid=(GRID,),
        in_specs=[
            pl.BlockSpec((NSC, RB, HG), lambda i: (0, i, 0)),
            pl.BlockSpec((RB, DM), lambda i: (i, 0)),
            pl.BlockSpec((1, DM), lambda i: (0, 0)),
            pl.BlockSpec((1, DM), lambda i: (0, 0)),
            pl.BlockSpec((DM, d_ff), lambda i: (0, 0)),
            pl.BlockSpec((1, d_ff), lambda i: (0, 0)),
            pl.BlockSpec((1, d_ff), lambda i: (0, 0)),
            pl.BlockSpec((d_ff, DM), lambda i: (0, 0)),
            pl.BlockSpec((1, DM), lambda i: (0, 0)),
        ],
        out_specs=pl.BlockSpec((RB, DM), lambda i: (i, 0)),
        out_shape=jax.ShapeDtypeStruct((N, DM), jnp.float32),
    )(ft2, x, g, b, w1, b1, al, w2, b2)


def kernel(x, params, edge_index):
    src = edge_index[0]
    dst = edge_index[1]
    z128 = jnp.zeros((NP, HG), jnp.float32)
    z16 = jnp.zeros((NP, LANE), jnp.float32)
    feat = x
    for p in params:
        q, kv = _qkv(feat, p['Wq'], p['Wk'], p['Wv'])
        ft2 = _EDGE_CALL(
            kv.reshape(NSC * NP, 2 * HG), q.reshape(NSC * NP, HG),
            src, dst, z128, z16)
        feat = _post(
            ft2.reshape(NSC, NP, HG), feat,
            p['g'].reshape(1, DM), p['b'].reshape(1, DM),
            p['W1'], p['b1'].reshape(1, -1), p['alpha'].reshape(1, -1),
            p['W2'], p['b2'].reshape(1, DM))
    return feat


# speedup vs baseline: 1.2419x; 1.1048x over previous
import functools

import jax
import jax.numpy as jnp
from jax import lax
from jax.experimental import pallas as pl
from jax.experimental.pallas import tpu as pltpu
from jax.experimental.pallas import tpu_sc as plsc

N = 10000
NP = 10240
E = 160000
DM = 256
HG = 128
NSC = 2
NSUB = 16
LANE = 16

C = 64
CH = C // 2
GH = CH // LANE
EPAD = 160768
EP = EPAD // NSUB
M = EP // CH
RN = NP // NSUB
RC = 64
NNORM = RN // RC

RB = 400
GRID = N // RB


def _qkv_body(x_ref, wq_ref, wk_ref, wv_ref, q_ref, k_ref, v_ref):
    xb = x_ref[...]
    for w_ref, o_ref in ((wq_ref, q_ref), (wk_ref, k_ref), (wv_ref, v_ref)):
        r = jnp.dot(xb, w_ref[...], preferred_element_type=jnp.float32)
        o_ref[0] = r[:, :HG]
        o_ref[1] = r[:, HG:]


def _qkv(x, wq, wk, wv):
    out = jax.ShapeDtypeStruct((NSC, NP, HG), jnp.float32)
    return pl.pallas_call(
        _qkv_body,
        grid=(GRID,),
        in_specs=[
            pl.BlockSpec((RB, DM), lambda i: (i, 0)),
            pl.BlockSpec((DM, DM), lambda i: (0, 0)),
            pl.BlockSpec((DM, DM), lambda i: (0, 0)),
            pl.BlockSpec((DM, DM), lambda i: (0, 0)),
        ],
        out_specs=[
            pl.BlockSpec((NSC, RB, HG), lambda i: (0, i, 0)),
            pl.BlockSpec((NSC, RB, HG), lambda i: (0, i, 0)),
            pl.BlockSpec((NSC, RB, HG), lambda i: (0, i, 0)),
        ],
        out_shape=[out, out, out],
    )(x, wq, wk, wv)


def _edge_body(k_hbm, q_hbm, v_hbm, src_hbm, dst_hbm, z128_hbm, z16_hbm,
               out_hbm,
               ft2_acc, den_acc,
               srcv, dstv, gsrcv, gdstv,
               krows, qrows, vrows, dstage, sem):
    c = lax.axis_index("c")
    s = lax.axis_index("s")
    cN = c * NP

    r0 = s * RN
    pltpu.sync_copy(z128_hbm.at[pl.ds(r0, RN)], ft2_acc.at[pl.ds(r0, RN)])
    pltpu.sync_copy(z16_hbm.at[pl.ds(r0, RN)], den_acc.at[pl.ds(r0, RN)])
    plsc.subcore_barrier()

    lanes = lax.iota(jnp.int32, LANE)
    base0 = s * EP
    zf = jnp.zeros((LANE,), jnp.float32)

    def dz(i, carry):
        dstage[i] = zf
        return carry
    lax.fori_loop(0, C, dz, 0)

    def fetch(t, half):
        base = base0 + t * CH
        pltpu.sync_copy(src_hbm.at[pl.ds(base, CH)], srcv.at[half])
        pltpu.sync_copy(dst_hbm.at[pl.ds(base, CH)], dstv.at[half])

        def gi(j, carry2):
            sl = pl.ds(j * LANE, LANE)
            gsrcv[half, sl] = srcv[half, sl] + cN
            gdstv[half, sl] = dstv[half, sl] + cN
            return carry2
        lax.fori_loop(0, CH // LANE, gi, 0)
        off = half * CH
        pltpu.async_copy(k_hbm.at[gsrcv.at[half]], krows.at[pl.ds(off, CH)], sem)
        pltpu.async_copy(q_hbm.at[gdstv.at[half]], qrows.at[pl.ds(off, CH)], sem)
        pltpu.async_copy(v_hbm.at[gsrcv.at[half]], vrows.at[pl.ds(off, CH)], sem)

    fetch(0, 0)

    def half_body(t, carry):
        half = jnp.bitwise_and(t, 1)
        off = half * CH
        pltpu.make_async_copy(
            k_hbm.at[pl.ds(0, CH)], krows.at[pl.ds(off, CH)], sem).wait()
        pltpu.make_async_copy(
            q_hbm.at[pl.ds(0, CH)], qrows.at[pl.ds(off, CH)], sem).wait()
        pltpu.make_async_copy(
            v_hbm.at[pl.ds(0, CH)], vrows.at[pl.ds(off, CH)], sem).wait()

        @pl.when(t + 1 < M)
        def _():
            fetch(t + 1, 1 - half)

        for g in range(GH):
            ridx = lanes + (g * LANE) + off

            def dot_d(d, accs):
                cb = jnp.bitwise_and(d + lanes, 31)
                out = []
                for h in range(4):
                    cvec = (h * 32) + cb
                    kv = plsc.load_gather(krows, [ridx, cvec])
                    qv = plsc.load_gather(qrows, [ridx, cvec])
                    out.append(accs[h] + kv * qv)
                return tuple(out)
            accs = lax.fori_loop(0, 32, dot_d, (zf, zf, zf, zf), unroll=4)
            evs = []
            for h in range(4):
                ev = jnp.exp(accs[h] * 0.0625)
                evs.append(ev)
                plsc.store_scatter(
                    dstage, [ridx, jnp.full((LANE,), h, jnp.int32)], ev)

            def scale_d(d, carry2):
                cb = jnp.bitwise_and(d + lanes, 31)
                for h in range(4):
                    cvec = (h * 32) + cb
                    vv = plsc.load_gather(vrows, [ridx, cvec])
                    plsc.store_scatter(qrows, [ridx, cvec], vv * evs[h])
                return carry2
            lax.fori_loop(0, 32, scale_d, 0, unroll=4)

        pltpu.sync_copy(qrows.at[pl.ds(off, CH)],
                        ft2_acc.at[dstv.at[half]], add=True)
        pltpu.sync_copy(dstage.at[pl.ds(off, CH)],
                        den_acc.at[dstv.at[half]], add=True)
        return carry
    lax.fori_loop(0, M, half_body, 0)
    plsc.subcore_barrier()

    def norm_chunk(t, carry):
        row = s * RN + t * RC
        pltpu.sync_copy(ft2_acc.at[pl.ds(row, RC)], krows)
        pltpu.sync_copy(den_acc.at[pl.ds(row, RC)], dstage)

        def row_body(r, carry2):
            dv = jnp.maximum(dstage[r], 1e-20)
            inv = 1.0 / dv
            for h in range(4):
                bv = jnp.full((LANE,), inv[h], jnp.float32)
                for j in (2 * h, 2 * h + 1):
                    sl = pl.ds(j * LANE, LANE)
                    krows[r, sl] = krows[r, sl] * bv
            return carry2
        lax.fori_loop(0, RC, row_body, 0)
        pltpu.sync_copy(krows, out_hbm.at[pl.ds(cN + row, RC)])
        return carry
    lax.fori_loop(0, NNORM, norm_chunk, 0)


def _make_edge_call():
    mesh = plsc.VectorSubcoreMesh(core_axis_name="c", subcore_axis_name="s")
    f32 = jnp.float32
    return pl.kernel(
        _edge_body,
        out_type=jax.ShapeDtypeStruct((NSC * NP, HG), f32),
        mesh=mesh,
        compiler_params=pltpu.CompilerParams(
            needs_layout_passes=False, use_tc_tiling_on_sc=False),
        scratch_types=[
            pltpu.VMEM_SHARED((NP, HG), f32),
            pltpu.VMEM_SHARED((NP, LANE), f32),
            pltpu.VMEM((2, CH), jnp.int32),
            pltpu.VMEM((2, CH), jnp.int32),
            pltpu.VMEM((2, CH), jnp.int32),
            pltpu.VMEM((2, CH), jnp.int32),
            pltpu.VMEM((C, HG), f32),
            pltpu.VMEM((C, HG), f32),
            pltpu.VMEM((C, HG), f32),
            pltpu.VMEM((C, LANE), f32),
            pltpu.SemaphoreType.DMA,
        ],
    )


_EDGE_CALL = _make_edge_call()


def _ln(x, g, b):
    mu = jnp.mean(x, axis=-1, keepdims=True)
    var = jnp.mean((x - mu) ** 2, axis=-1, keepdims=True)
    return (x - mu) / jnp.sqrt(var + 1e-5) * g + b


def _post_body(ft2_ref, x_ref, g_ref, b_ref, w1_ref, b1_ref, al_ref,
               w2_ref, b2_ref, o_ref):
    g = g_ref[...]
    b = b_ref[...]
    rst = jnp.concatenate([ft2_ref[0], ft2_ref[1]], axis=1) + x_ref[...]
    rst = _ln(rst, g, b)
    h = jnp.dot(rst, w1_ref[...], preferred_element_type=jnp.float32)
    h = h + b1_ref[...]
    h = jnp.where(h > 0, h, al_ref[...] * h)
    ffn = jnp.dot(h, w2_ref[...], preferred_element_type=jnp.float32)
    ffn = ffn + b2_ref[...]
    o_ref[...] = _ln(rst + ffn, g, b)


def _post(ft2, x, g, b, w1, b1, al, w2, b2):
    d_ff = w1.shape[1]
    return pl.pallas_call(
        _post_body,
        grid=(GRID,),
        in_specs=[
            pl.BlockSpec((NSC, RB, HG), lambda i: (0, i, 0)),
            pl.BlockSpec((RB, DM), lambda i: (i, 0)),
            pl.BlockSpec((1, DM), lambda i: (0, 0)),
            pl.BlockSpec((1, DM), lambda i: (0, 0)),
            pl.BlockSpec((DM, d_ff), lambda i: (0, 0)),
            pl.BlockSpec((1, d_ff), lambda i: (0, 0)),
            pl.BlockSpec((1, d_ff), lambda i: (0, 0)),
            pl.BlockSpec((d_ff, DM), lambda i: (0, 0)),
            pl.BlockSpec((1, DM), lambda i: (0, 0)),
        ],
        out_specs=pl.BlockSpec((RB, DM), lambda i: (i, 0)),
        out_shape=jax.ShapeDtypeStruct((N, DM), jnp.float32),
    )(ft2, x, g, b, w1, b1, al, w2, b2)


def kernel(x, params, edge_index):
    src = jnp.concatenate(
        [edge_index[0], jnp.zeros((EPAD - E,), jnp.int32)])
    dst = jnp.concatenate(
        [edge_index[1], jnp.full((EPAD - E,), NP - 1, jnp.int32)])
    z128 = jnp.zeros((NP, HG), jnp.float32)
    z16 = jnp.zeros((NP, LANE), jnp.float32)
    feat = x
    for p in params:
        q, k, v = _qkv(feat, p['Wq'], p['Wk'], p['Wv'])
        ft2 = _EDGE_CALL(
            k.reshape(NSC * NP, HG), q.reshape(NSC * NP, HG),
            v.reshape(NSC * NP, HG), src, dst, z128, z16)
        feat = _post(
            ft2.reshape(NSC, NP, HG), feat,
            p['g'].reshape(1, DM), p['b'].reshape(1, DM),
            p['W1'], p['b1'].reshape(1, -1), p['alpha'].reshape(1, -1),
            p['W2'], p['b2'].reshape(1, DM))
    return feat


# speedup vs baseline: 1.6084x; 1.2952x over previous
import functools

import jax
import jax.numpy as jnp
from jax import lax
from jax.experimental import pallas as pl
from jax.experimental.pallas import tpu as pltpu
from jax.experimental.pallas import tpu_sc as plsc

N = 10000
NP = 10240
E = 160000
DM = 256
HG = 128
NSC = 2
NSUB = 16
LANE = 16

C = 64
CH = C // 2
GH = CH // LANE
EPAD = 160768
EPAD2 = 163840
EP = EPAD // NSUB
M = EP // CH
NB = 16
RN = NP // NSUB
RC = 64
NNORM = RN // RC

RB = 400
GRID = N // RB


def _qkv_body(x_ref, wq_ref, wk_ref, wv_ref, q_ref, k_ref, v_ref):
    xb = x_ref[...]
    for w_ref, o_ref in ((wq_ref, q_ref), (wk_ref, k_ref), (wv_ref, v_ref)):
        r = jnp.dot(xb, w_ref[...], preferred_element_type=jnp.float32)
        o_ref[0] = r[:, :HG]
        o_ref[1] = r[:, HG:]


def _qkv(x, wq, wk, wv):
    out = jax.ShapeDtypeStruct((NSC, NP, HG), jnp.float32)
    return pl.pallas_call(
        _qkv_body,
        grid=(GRID,),
        in_specs=[
            pl.BlockSpec((RB, DM), lambda i: (i, 0)),
            pl.BlockSpec((DM, DM), lambda i: (0, 0)),
            pl.BlockSpec((DM, DM), lambda i: (0, 0)),
            pl.BlockSpec((DM, DM), lambda i: (0, 0)),
        ],
        out_specs=[
            pl.BlockSpec((NSC, RB, HG), lambda i: (0, i, 0)),
            pl.BlockSpec((NSC, RB, HG), lambda i: (0, i, 0)),
            pl.BlockSpec((NSC, RB, HG), lambda i: (0, i, 0)),
        ],
        out_shape=[out, out, out],
    )(x, wq, wk, wv)


def _edge_body(k_hbm, q_hbm, v_hbm, src_hbm, dst_hbm, z128_hbm, z16_hbm,
               out_hbm,
               ft2_acc, den_acc,
               srcv, dstv, gsrcv, gdstv,
               krows, qrows, vrows, dstage, sem):
    c = lax.axis_index("c")
    s = lax.axis_index("s")
    cN = c * NP

    r0 = s * RN
    pltpu.sync_copy(z128_hbm.at[pl.ds(r0, RN)], ft2_acc.at[pl.ds(r0, RN)])
    pltpu.sync_copy(z16_hbm.at[pl.ds(r0, RN)], den_acc.at[pl.ds(r0, RN)])
    plsc.subcore_barrier()

    lanes = lax.iota(jnp.int32, LANE)
    base0 = s * EP
    zf = jnp.zeros((LANE,), jnp.float32)

    def dz(i, carry):
        dstage[i] = zf
        return carry
    lax.fori_loop(0, C, dz, 0)

    def refill(b):
        b1 = jnp.bitwise_and(b, 1)
        rbase = s * M + b * NB
        pltpu.sync_copy(src_hbm.at[pl.ds(rbase, NB)], srcv.at[b1])
        pltpu.sync_copy(dst_hbm.at[pl.ds(rbase, NB)], dstv.at[b1])

        def gi(r, carry2):
            for j in range(CH // LANE):
                sl = pl.ds(j * LANE, LANE)
                gsrcv[b1, r, sl] = srcv[b1, r, sl] + cN
                gdstv[b1, r, sl] = dstv[b1, r, sl] + cN
            return carry2
        lax.fori_loop(0, NB, gi, 0)

    def fetch(t, half):
        blk = jnp.right_shift(t, 4)
        buf = jnp.bitwise_and(blk, 1)
        row = jnp.bitwise_and(t, NB - 1)
        off = half * CH
        pltpu.async_copy(k_hbm.at[gsrcv.at[buf, row]],
                         krows.at[pl.ds(off, CH)], sem)
        pltpu.async_copy(q_hbm.at[gdstv.at[buf, row]],
                         qrows.at[pl.ds(off, CH)], sem)
        pltpu.async_copy(v_hbm.at[gsrcv.at[buf, row]],
                         vrows.at[pl.ds(off, CH)], sem)

    refill(jnp.int32(0))
    fetch(jnp.int32(0), 0)

    def half_body(t, carry):
        half = jnp.bitwise_and(t, 1)
        off = half * CH
        blk = jnp.right_shift(t, 4)
        buf = jnp.bitwise_and(blk, 1)
        row = jnp.bitwise_and(t, NB - 1)
        pltpu.make_async_copy(
            k_hbm.at[pl.ds(0, CH)], krows.at[pl.ds(off, CH)], sem).wait()
        pltpu.make_async_copy(
            q_hbm.at[pl.ds(0, CH)], qrows.at[pl.ds(off, CH)], sem).wait()
        pltpu.make_async_copy(
            v_hbm.at[pl.ds(0, CH)], vrows.at[pl.ds(off, CH)], sem).wait()

        @pl.when(row == 0)
        def _():
            refill(blk + 1)

        @pl.when(t + 1 < M)
        def _():
            fetch(t + 1, 1 - half)

        for g in range(GH):
            ridx = lanes + (g * LANE) + off

            def dot_d(d, accs):
                cb = jnp.bitwise_and(d + lanes, 31)
                out = []
                for h in range(4):
                    cvec = (h * 32) + cb
                    kv = plsc.load_gather(krows, [ridx, cvec])
                    qv = plsc.load_gather(qrows, [ridx, cvec])
                    out.append(accs[h] + kv * qv)
                return tuple(out)
            accs = lax.fori_loop(0, 32, dot_d, (zf, zf, zf, zf), unroll=4)
            evs = []
            for h in range(4):
                ev = jnp.exp(accs[h] * 0.0625)
                evs.append(ev)
                plsc.store_scatter(
                    dstage, [ridx, jnp.full((LANE,), h, jnp.int32)], ev)

            def scale_d(d, carry2):
                cb = jnp.bitwise_and(d + lanes, 31)
                for h in range(4):
                    cvec = (h * 32) + cb
                    vv = plsc.load_gather(vrows, [ridx, cvec])
                    plsc.store_scatter(qrows, [ridx, cvec], vv * evs[h])
                return carry2
            lax.fori_loop(0, 32, scale_d, 0, unroll=4)

        pltpu.sync_copy(qrows.at[pl.ds(off, CH)],
                        ft2_acc.at[dstv.at[buf, row]], add=True)
        pltpu.sync_copy(dstage.at[pl.ds(off, CH)],
                        den_acc.at[dstv.at[buf, row]], add=True)
        return carry
    lax.fori_loop(0, M, half_body, 0)
    plsc.subcore_barrier()

    def norm_chunk(t, carry):
        row = s * RN + t * RC
        pltpu.sync_copy(ft2_acc.at[pl.ds(row, RC)], krows)
        pltpu.sync_copy(den_acc.at[pl.ds(row, RC)], dstage)

        def row_body(r, carry2):
            dv = jnp.maximum(dstage[r], 1e-20)
            inv = 1.0 / dv
            for h in range(4):
                bv = jnp.full((LANE,), inv[h], jnp.float32)
                for j in (2 * h, 2 * h + 1):
                    sl = pl.ds(j * LANE, LANE)
                    krows[r, sl] = krows[r, sl] * bv
            return carry2
        lax.fori_loop(0, RC, row_body, 0)
        pltpu.sync_copy(krows, out_hbm.at[pl.ds(cN + row, RC)])
        return carry
    lax.fori_loop(0, NNORM, norm_chunk, 0)


def _make_edge_call():
    mesh = plsc.VectorSubcoreMesh(core_axis_name="c", subcore_axis_name="s")
    f32 = jnp.float32
    return pl.kernel(
        _edge_body,
        out_type=jax.ShapeDtypeStruct((NSC * NP, HG), f32),
        mesh=mesh,
        compiler_params=pltpu.CompilerParams(
            needs_layout_passes=False, use_tc_tiling_on_sc=False),
        scratch_types=[
            pltpu.VMEM_SHARED((NP, HG), f32),
            pltpu.VMEM_SHARED((NP, LANE), f32),
            pltpu.VMEM((2, NB, CH), jnp.int32),
            pltpu.VMEM((2, NB, CH), jnp.int32),
            pltpu.VMEM((2, NB, CH), jnp.int32),
            pltpu.VMEM((2, NB, CH), jnp.int32),
            pltpu.VMEM((C, HG), f32),
            pltpu.VMEM((C, HG), f32),
            pltpu.VMEM((C, HG), f32),
            pltpu.VMEM((C, LANE), f32),
            pltpu.SemaphoreType.DMA,
        ],
    )


_EDGE_CALL = _make_edge_call()


def _ln(x, g, b):
    mu = jnp.mean(x, axis=-1, keepdims=True)
    var = jnp.mean((x - mu) ** 2, axis=-1, keepdims=True)
    return (x - mu) / jnp.sqrt(var + 1e-5) * g + b


def _post_body(ft2_ref, x_ref, g_ref, b_ref, w1_ref, b1_ref, al_ref,
               w2_ref, b2_ref, o_ref):
    g = g_ref[...]
    b = b_ref[...]
    rst = jnp.concatenate([ft2_ref[0], ft2_ref[1]], axis=1) + x_ref[...]
    rst = _ln(rst, g, b)
    h = jnp.dot(rst, w1_ref[...], preferred_element_type=jnp.float32)
    h = h + b1_ref[...]
    h = jnp.where(h > 0, h, al_ref[...] * h)
    ffn = jnp.dot(h, w2_ref[...], preferred_element_type=jnp.float32)
    ffn = ffn + b2_ref[...]
    o_ref[...] = _ln(rst + ffn, g, b)


def _post(ft2, x, g, b, w1, b1, al, w2, b2):
    d_ff = w1.shape[1]
    return pl.pallas_call(
        _post_body,
        grid=(GRID,),
        in_specs=[
            pl.BlockSpec((NSC, RB, HG), lambda i: (0, i, 0)),
            pl.BlockSpec((RB, DM), lambda i: (i, 0)),
            pl.BlockSpec((1, DM), lambda i: (0, 0)),
            pl.BlockSpec((1, DM), lambda i: (0, 0)),
            pl.BlockSpec((DM, d_ff), lambda i: (0, 0)),
            pl.BlockSpec((1, d_ff), lambda i: (0, 0)),
            pl.BlockSpec((1, d_ff), lambda i: (0, 0)),
            pl.BlockSpec((d_ff, DM), lambda i: (0, 0)),
            pl.BlockSpec((1, DM), lambda i: (0, 0)),
        ],
        out_specs=pl.BlockSpec((RB, DM), lambda i: (i, 0)),
        out_shape=jax.ShapeDtypeStruct((N, DM), jnp.float32),
    )(ft2, x, g, b, w1, b1, al, w2, b2)


def kernel(x, params, edge_index):
    src = jnp.concatenate(
        [edge_index[0], jnp.zeros((EPAD2 - E,), jnp.int32)]
    ).reshape(EPAD2 // CH, CH)
    dst = jnp.concatenate(
        [edge_index[1], jnp.full((EPAD2 - E,), NP - 1, jnp.int32)]
    ).reshape(EPAD2 // CH, CH)
    z128 = jnp.zeros((NP, HG), jnp.float32)
    z16 = jnp.zeros((NP, LANE), jnp.float32)
    feat = x
    for p in params:
        q, k, v = _qkv(feat, p['Wq'], p['Wk'], p['Wv'])
        ft2 = _EDGE_CALL(
            k.reshape(NSC * NP, HG), q.reshape(NSC * NP, HG),
            v.reshape(NSC * NP, HG), src, dst, z128, z16)
        feat = _post(
            ft2.reshape(NSC, NP, HG), feat,
            p['g'].reshape(1, DM), p['b'].reshape(1, DM),
            p['W1'], p['b1'].reshape(1, -1), p['alpha'].reshape(1, -1),
            p['W2'], p['b2'].reshape(1, DM))
    return feat
